# baseline (speedup 1.0000x reference)
import functools

import jax
import jax.numpy as jnp
from jax import lax
from jax.experimental import pallas as pl
from jax.experimental.pallas import tpu as pltpu
from jax.experimental.pallas import tpu_sc as plsc

N = 10000
E = 320000
D = 128
ALPHA = 0.1

NC = 2
NS = 16
NW = NC * NS
EPT = E // NW
K = 40
NCHUNK = EPT // K
RPT = 624
TAIL = N - NS * RPT
NSUM = 10240
SPT = NSUM // NS
BB = 1024
BR = 2000

_ZCHUNKS = (40,) * 15 + (24,)


def _mm_body(x_ref, w_ref, attn_ref, h_ref, a_ref):
    h = jnp.dot(x_ref[...], w_ref[...], preferred_element_type=jnp.float32)
    h_ref[...] = h
    a_ref[...] = jnp.dot(h, attn_ref[...], preferred_element_type=jnp.float32)


def _fin_body(p_ref, r_ref, o_ref):
    tot = p_ref[0] + p_ref[1]
    rs = r_ref[0, pl.ds(0, N)] + r_ref[1, pl.ds(0, N)]
    o = tot / (rs[:, None] + 1e-30)
    o_ref[...] = jnp.where(o >= 0, o, ALPHA * o)


def _edge_body(h_hbm, a1_hbm, a2_hbm, ei_hbm, p_hbm, r_hbm,
               a1_v, a2_v, zb_v, sb_v, db_v, si3, w3, rows3,
               acc_sh, rsum_sh, gsem0, gsem1, gsem2, ssem0, ssem1, ssem2):
    gsems = (gsem0, gsem1, gsem2)
    ssems = (ssem0, ssem1, ssem2)
    c = lax.axis_index("c")
    s = lax.axis_index("s")
    wid = c * NS + s

    zero16 = jnp.zeros((16,), jnp.float32)

    def _zrows(i, _):
        rows3[0, i // 8, pl.ds((i % 8) * 16, 16)] = zero16
        return 0
    lax.fori_loop(0, K * (D // 16), _zrows, 0)

    def _zb(i, _):
        zb_v[pl.ds(i * 16, 16)] = zero16
        return 0
    lax.fori_loop(0, BB // 16, _zb, 0)

    base_row = pl.multiple_of(s * RPT, 8)
    off = 0
    for n in _ZCHUNKS:
        pltpu.sync_copy(rows3.at[0, pl.ds(0, n)], acc_sh.at[pl.ds(base_row + off, n)])
        off += n

    @pl.when(s == NS - 1)
    def _zero_tail():
        pltpu.sync_copy(rows3.at[0, pl.ds(0, TAIL)], acc_sh.at[pl.ds(NS * RPT, TAIL)])

    pltpu.sync_copy(zb_v.at[pl.ds(0, SPT)],
                    rsum_sh.at[pl.ds(pl.multiple_of(s * SPT, 8), SPT)])

    pltpu.sync_copy(a1_hbm, a1_v)
    pltpu.sync_copy(a2_hbm, a2_v)

    lane = jnp.arange(16, dtype=jnp.int32)

    def _mx(ref):
        def body(i, m):
            return jnp.maximum(m, ref[pl.ds(i * 16, 16)])
        m = lax.fori_loop(0, N // 16, body, jnp.full((16,), -1e30, jnp.float32))
        for step in (8, 4, 2, 1):
            zb_v[pl.ds(0, 16)] = m
            m = jnp.maximum(m, plsc.load_gather(zb_v, [lane ^ step]))
        return m
    M = _mx(a1_v) + _mx(a2_v)

    ebase = wid * EPT

    def _goff(g):
        return pl.multiple_of(g * K, 8)

    def _prefetch(g, b):
        pltpu.async_copy(h_hbm.at[db_v.at[pl.ds(_goff(g), K)]],
                         rows3.at[b], gsems[b])

    def _drain_scatters(b):
        pltpu.make_async_copy(rows3.at[b], acc_sh.at[si3.at[b]], ssems[b]).wait()
        pltpu.make_async_copy(w3.at[b], rsum_sh.at[si3.at[b]], ssems[b]).wait()

    woffs = list(range(0, K - 15, 16))
    if K % 16:
        woffs.append(K - 16)

    def _do_chunk(g, b):
        pltpu.make_async_copy(h_hbm.at[db_v.at[pl.ds(_goff(g), K)]],
                              rows3.at[b], gsems[b]).wait()
        bvec = jnp.full((16,), b, jnp.int32)

        for o in woffs:
            sv = sb_v[pl.ds(pl.multiple_of(g * K + o, 8), 16)]
            dv = db_v[pl.ds(pl.multiple_of(g * K + o, 8), 16)]
            si3[b, pl.ds(o, 16)] = sv
            val = plsc.load_gather(a1_v, [sv]) + plsc.load_gather(a2_v, [dv])
            val = jnp.where(val >= 0, val, ALPHA * val)
            w3[b, pl.ds(o, 16)] = jnp.exp(val - M)

        @plsc.parallel_loop(0, K, unroll=8)
        def _scale(e):
            wvec = plsc.load_gather(w3, [bvec, jnp.zeros((16,), jnp.int32) + e])
            for q in range(D // 16):
                rows3[b, e, pl.ds(q * 16, 16)] = rows3[b, e, pl.ds(q * 16, 16)] * wvec

        pltpu.async_copy(rows3.at[b], acc_sh.at[si3.at[b]], ssems[b], add=True)
        pltpu.async_copy(w3.at[b], rsum_sh.at[si3.at[b]], ssems[b], add=True)

    NCH = NCHUNK // 2
    NT = (NCH - 4) // 3
    EHALF = EPT // 2

    def _main(t, _):
        for j in range(3):
            g = t * 3 + j
            b = j
            _do_chunk(g, b)
            b2 = (j + 2) % 3
            if j == 0:
                @pl.when(t > 0)
                def _():
                    _drain_scatters(b2)
            else:
                _drain_scatters(b2)
            _prefetch(g + 2, b2)
        return 0

    first = True
    for half in (0, 1):
        hb = ebase + half * EHALF
        pltpu.sync_copy(ei_hbm.at[pl.ds(pl.multiple_of(hb, 8), EHALF)], sb_v)
        pltpu.sync_copy(ei_hbm.at[pl.ds(pl.multiple_of(E + hb, 8), EHALF)], db_v)
        _prefetch(0, 0)
        _prefetch(1, 1)
        if first:
            plsc.subcore_barrier()
            first = False
        lax.fori_loop(0, NT, _main, 0)
        for g in range(3 * NT, NCH):
            b = g % 3
            _do_chunk(g, b)
            _drain_scatters((g + 2) % 3)
            if g + 2 < NCH:
                _prefetch(g + 2, (g + 2) % 3)
        _drain_scatters((NCH - 1) % 3)

    plsc.subcore_barrier()

    off = 0
    for n in _ZCHUNKS:
        pltpu.sync_copy(acc_sh.at[pl.ds(base_row + off, n)], rows3.at[0, pl.ds(0, n)])
        pltpu.sync_copy(rows3.at[0, pl.ds(0, n)], p_hbm.at[c, pl.ds(base_row + off, n)])
        off += n

    @pl.when(s == NS - 1)
    def _wb_tail():
        pltpu.sync_copy(acc_sh.at[pl.ds(NS * RPT, TAIL)], rows3.at[0, pl.ds(0, TAIL)])
        pltpu.sync_copy(rows3.at[0, pl.ds(0, TAIL)], p_hbm.at[c, pl.ds(NS * RPT, TAIL)])

    @pl.when(s == 0)
    def _wb_rsum():
        rbase = pl.multiple_of(c * NSUM, 8)
        for kk in range(NSUM // BB):
            pltpu.sync_copy(rsum_sh.at[pl.ds(kk * BB, BB)], zb_v)
            pltpu.sync_copy(zb_v, r_hbm.at[pl.ds(rbase + kk * BB, BB)])


_edge_kernel = functools.partial(
    pl.kernel,
    out_type=(
        jax.ShapeDtypeStruct((NC, N, D), jnp.float32),
        jax.ShapeDtypeStruct((NC * NSUM,), jnp.float32),
    ),
    mesh=plsc.VectorSubcoreMesh(core_axis_name="c", subcore_axis_name="s"),
    scratch_types=[
        pltpu.VMEM((N,), jnp.float32),
        pltpu.VMEM((N,), jnp.float32),
        pltpu.VMEM((BB,), jnp.float32),
        pltpu.VMEM((EPT // 2,), jnp.int32),
        pltpu.VMEM((EPT // 2,), jnp.int32),
        pltpu.VMEM((3, K), jnp.int32),
        pltpu.VMEM((3, K), jnp.float32),
        pltpu.VMEM((3, K, D), jnp.float32),
        pltpu.MemorySpace.VMEM_SHARED((N, D), jnp.float32),
        pltpu.MemorySpace.VMEM_SHARED((NSUM,), jnp.float32),
        pltpu.SemaphoreType.DMA,
        pltpu.SemaphoreType.DMA,
        pltpu.SemaphoreType.DMA,
        pltpu.SemaphoreType.DMA,
        pltpu.SemaphoreType.DMA,
        pltpu.SemaphoreType.DMA,
    ],
    compiler_params=pltpu.CompilerParams(needs_layout_passes=False),
)(_edge_body)


@jax.jit
def kernel(x, edge_index, W, attn):
    attn_rs = attn.reshape(2, D).T

    h, a = pl.pallas_call(
        _mm_body,
        grid=(N // BR,),
        in_specs=[
            pl.BlockSpec((BR, D), lambda i: (i, 0)),
            pl.BlockSpec((D, D), lambda i: (0, 0)),
            pl.BlockSpec((D, 2), lambda i: (0, 0)),
        ],
        out_specs=[
            pl.BlockSpec((BR, D), lambda i: (i, 0)),
            pl.BlockSpec((BR, 2), lambda i: (i, 0)),
        ],
        out_shape=[
            jax.ShapeDtypeStruct((N, D), jnp.float32),
            jax.ShapeDtypeStruct((N, 2), jnp.float32),
        ],
    )(x, W, attn_rs)

    p, r = _edge_kernel(h, a[:, 0], a[:, 1], edge_index.reshape(2 * E))
    r = r.reshape(NC, NSUM)

    out = pl.pallas_call(
        _fin_body,
        out_shape=jax.ShapeDtypeStruct((N, D), jnp.float32),
    )(p, r)
    return out


# speedup vs baseline: 1.2302x; 1.2302x over previous
import functools

import jax
import jax.numpy as jnp
from jax import lax
from jax.experimental import pallas as pl
from jax.experimental.pallas import tpu as pltpu
from jax.experimental.pallas import tpu_sc as plsc

N = 10000
E = 320000
D = 128
ALPHA = 0.1

NC = 2
NS = 16
NW = NC * NS
EPT = E // NW
K = 40
NCHUNK = EPT // K
RPT = 624
TAIL = N - NS * RPT
NSUM = 10240
SPT = NSUM // NS
BB = 1024
BR = 2000

_ZCHUNKS = (40,) * 15 + (24,)


def _mm_body(x_ref, w_ref, attn_ref, h_ref, a_ref):
    h = jnp.dot(x_ref[...], w_ref[...], preferred_element_type=jnp.float32)
    h_ref[...] = h
    a_ref[...] = jnp.dot(h, attn_ref[...], preferred_element_type=jnp.float32)


def _fin_body(p_ref, r_ref, o_ref):
    tot = p_ref[0] + p_ref[1]
    rs = r_ref[0, pl.ds(0, N)] + r_ref[1, pl.ds(0, N)]
    o = tot / (rs[:, None] + 1e-30)
    o_ref[...] = jnp.where(o >= 0, o, ALPHA * o)


def _edge_body(h_hbm, a1_hbm, a2_hbm, ei_hbm, p_hbm, r_hbm,
               a1_v, a2_v, zb_v, db_v, si4, w4, rows4,
               acc_sh, rsum_sh,
               gsem0, gsem1, gsem2, gsem3, ssem0, ssem1, ssem2, ssem3):
    gsems = (gsem0, gsem1, gsem2, gsem3)
    ssems = (ssem0, ssem1, ssem2, ssem3)
    c = lax.axis_index("c")
    s = lax.axis_index("s")
    wid = c * NS + s

    zero16 = jnp.zeros((16,), jnp.float32)

    def _zrows(i, _):
        rows4[0, i // 8, pl.ds((i % 8) * 16, 16)] = zero16
        return 0
    lax.fori_loop(0, K * (D // 16), _zrows, 0)

    def _zb(i, _):
        zb_v[pl.ds(i * 16, 16)] = zero16
        return 0
    lax.fori_loop(0, BB // 16, _zb, 0)

    base_row = pl.multiple_of(s * RPT, 8)
    off = 0
    for n in _ZCHUNKS:
        pltpu.sync_copy(rows4.at[0, pl.ds(0, n)], acc_sh.at[pl.ds(base_row + off, n)])
        off += n

    @pl.when(s == NS - 1)
    def _zero_tail():
        pltpu.sync_copy(rows4.at[0, pl.ds(0, TAIL)], acc_sh.at[pl.ds(NS * RPT, TAIL)])

    pltpu.sync_copy(zb_v.at[pl.ds(0, SPT)],
                    rsum_sh.at[pl.ds(pl.multiple_of(s * SPT, 8), SPT)])

    pltpu.sync_copy(a1_hbm, a1_v)
    pltpu.sync_copy(a2_hbm, a2_v)

    lane = jnp.arange(16, dtype=jnp.int32)

    def _mx(ref):
        def body(i, m):
            return jnp.maximum(m, ref[pl.ds(i * 16, 16)])
        m = lax.fori_loop(0, N // 16, body, jnp.full((16,), -1e30, jnp.float32))
        for step in (8, 4, 2, 1):
            zb_v[pl.ds(0, 16)] = m
            m = jnp.maximum(m, plsc.load_gather(zb_v, [lane ^ step]))
        return m
    M = _mx(a1_v) + _mx(a2_v)

    ebase = wid * EPT

    def _goff(g):
        return pl.multiple_of(g * K, 8)

    def _prefetch(hb, g, b):
        pltpu.async_copy(ei_hbm.at[pl.ds(pl.multiple_of(hb + g * K, 8), K)],
                         si4.at[b], gsems[b])
        pltpu.async_copy(h_hbm.at[db_v.at[pl.ds(_goff(g), K)]],
                         rows4.at[b], gsems[b])

    def _drain_scatters(b):
        pltpu.make_async_copy(rows4.at[b], acc_sh.at[si4.at[b]], ssems[b]).wait()
        pltpu.make_async_copy(w4.at[b], rsum_sh.at[si4.at[b]], ssems[b]).wait()

    woffs = list(range(0, K - 15, 16))
    if K % 16:
        woffs.append(K - 16)

    def _do_chunk(hb, g, b):
        pltpu.make_async_copy(ei_hbm.at[pl.ds(pl.multiple_of(hb + g * K, 8), K)],
                              si4.at[b], gsems[b]).wait()
        pltpu.make_async_copy(h_hbm.at[db_v.at[pl.ds(_goff(g), K)]],
                              rows4.at[b], gsems[b]).wait()
        bvec = jnp.full((16,), b, jnp.int32)

        for o in woffs:
            sv = si4[b, pl.ds(o, 16)]
            dv = db_v[pl.ds(pl.multiple_of(g * K + o, 8), 16)]
            val = plsc.load_gather(a1_v, [sv]) + plsc.load_gather(a2_v, [dv])
            val = jnp.where(val >= 0, val, ALPHA * val)
            w4[b, pl.ds(o, 16)] = jnp.exp(val - M)

        @plsc.parallel_loop(0, K, unroll=4)
        def _scale(e):
            wvec = plsc.load_gather(w4, [bvec, jnp.zeros((16,), jnp.int32) + e])
            for q in range(D // 16):
                rows4[b, e, pl.ds(q * 16, 16)] = rows4[b, e, pl.ds(q * 16, 16)] * wvec

        pltpu.async_copy(rows4.at[b], acc_sh.at[si4.at[b]], ssems[b], add=True)
        pltpu.async_copy(w4.at[b], rsum_sh.at[si4.at[b]], ssems[b], add=True)

    NCH = NCHUNK // 2
    NT = (NCH - 5) // 4
    EHALF = EPT // 2

    def _mk_main(hb):
        def _main(t, _):
            for j in range(4):
                g = t * 4 + j
                b = j
                _do_chunk(hb, g, b)
                b2 = (j + 3) % 4
                if j == 0:
                    @pl.when(t > 0)
                    def _():
                        _drain_scatters(b2)
                else:
                    _drain_scatters(b2)
                _prefetch(hb, g + 3, b2)
            return 0
        return _main

    first = True
    for half in (0, 1):
        hb = ebase + half * EHALF
        pltpu.sync_copy(ei_hbm.at[pl.ds(pl.multiple_of(E + hb, 8), EHALF)], db_v)
        _prefetch(hb, 0, 0)
        _prefetch(hb, 1, 1)
        _prefetch(hb, 2, 2)
        if first:
            plsc.subcore_barrier()
            first = False
        lax.fori_loop(0, NT, _mk_main(hb), 0)
        for g in range(4 * NT, NCH):
            b = g % 4
            _do_chunk(hb, g, b)
            _drain_scatters((g + 3) % 4)
            if g + 3 < NCH:
                _prefetch(hb, g + 3, (g + 3) % 4)
        _drain_scatters((NCH - 1) % 4)

    plsc.subcore_barrier()

    off = 0
    for n in _ZCHUNKS:
        pltpu.sync_copy(acc_sh.at[pl.ds(base_row + off, n)], rows4.at[0, pl.ds(0, n)])
        pltpu.sync_copy(rows4.at[0, pl.ds(0, n)], p_hbm.at[c, pl.ds(base_row + off, n)])
        off += n

    @pl.when(s == NS - 1)
    def _wb_tail():
        pltpu.sync_copy(acc_sh.at[pl.ds(NS * RPT, TAIL)], rows4.at[0, pl.ds(0, TAIL)])
        pltpu.sync_copy(rows4.at[0, pl.ds(0, TAIL)], p_hbm.at[c, pl.ds(NS * RPT, TAIL)])

    @pl.when(s == 0)
    def _wb_rsum():
        rbase = pl.multiple_of(c * NSUM, 8)
        for kk in range(NSUM // BB):
            pltpu.sync_copy(rsum_sh.at[pl.ds(kk * BB, BB)], zb_v)
            pltpu.sync_copy(zb_v, r_hbm.at[pl.ds(rbase + kk * BB, BB)])


_edge_kernel = functools.partial(
    pl.kernel,
    out_type=(
        jax.ShapeDtypeStruct((NC, N, D), jnp.float32),
        jax.ShapeDtypeStruct((NC * NSUM,), jnp.float32),
    ),
    mesh=plsc.VectorSubcoreMesh(core_axis_name="c", subcore_axis_name="s"),
    scratch_types=[
        pltpu.VMEM((N,), jnp.float32),
        pltpu.VMEM((N,), jnp.float32),
        pltpu.VMEM((BB,), jnp.float32),
        pltpu.VMEM((EPT // 2,), jnp.int32),
        pltpu.VMEM((4, K), jnp.int32),
        pltpu.VMEM((4, K), jnp.float32),
        pltpu.VMEM((4, K, D), jnp.float32),
        pltpu.MemorySpace.VMEM_SHARED((N, D), jnp.float32),
        pltpu.MemorySpace.VMEM_SHARED((NSUM,), jnp.float32),
        pltpu.SemaphoreType.DMA,
        pltpu.SemaphoreType.DMA,
        pltpu.SemaphoreType.DMA,
        pltpu.SemaphoreType.DMA,
        pltpu.SemaphoreType.DMA,
        pltpu.SemaphoreType.DMA,
        pltpu.SemaphoreType.DMA,
        pltpu.SemaphoreType.DMA,
    ],
    compiler_params=pltpu.CompilerParams(needs_layout_passes=False),
)(_edge_body)


@jax.jit
def kernel(x, edge_index, W, attn):
    attn_rs = attn.reshape(2, D).T

    h, a = pl.pallas_call(
        _mm_body,
        grid=(N // BR,),
        in_specs=[
            pl.BlockSpec((BR, D), lambda i: (i, 0)),
            pl.BlockSpec((D, D), lambda i: (0, 0)),
            pl.BlockSpec((D, 2), lambda i: (0, 0)),
        ],
        out_specs=[
            pl.BlockSpec((BR, D), lambda i: (i, 0)),
            pl.BlockSpec((BR, 2), lambda i: (i, 0)),
        ],
        out_shape=[
            jax.ShapeDtypeStruct((N, D), jnp.float32),
            jax.ShapeDtypeStruct((N, 2), jnp.float32),
        ],
    )(x, W, attn_rs)

    p, r = _edge_kernel(h, a[:, 0], a[:, 1], edge_index.reshape(2 * E))
    r = r.reshape(NC, NSUM)

    out = pl.pallas_call(
        _fin_body,
        out_shape=jax.ShapeDtypeStruct((N, D), jnp.float32),
    )(p, r)
    return out
